# Initial kernel scaffold; baseline (speedup 1.0000x reference)
import jax
import jax.numpy as jnp
from jax.experimental import pallas as pl


def kernel(features, edge_index, W1, b1, W2, b2):
    raise NotImplementedError("write your pallas kernel here")


# speedup vs baseline: 8.2567x; 8.2567x over previous
import functools

import jax
import jax.numpy as jnp
from jax import lax
from jax.experimental import pallas as pl
from jax.experimental.pallas import tpu as pltpu
from jax.experimental.pallas import tpu_sc as plsc

N = 10000
E = 320000
D = 128
H = 64
C = 40
CP = 48
NC = 2
NS = 16
NT = NC * NS
B = 128
NCHUNK = E // B
ROWS_PER_SUB = 632
N_PAD = NS * ROWS_PER_SUB

_mesh = plsc.VectorSubcoreMesh(core_axis_name="c", subcore_axis_name="s")
_sc_params = pltpu.CompilerParams(use_tc_tiling_on_sc=False)


def _make_spmm(Wd):
  @functools.partial(
      pl.kernel,
      out_type=jax.ShapeDtypeStruct((NC, N_PAD, Wd), jnp.float32),
      mesh=_mesh,
      scratch_types=[
          pltpu.VMEM((B,), jnp.int32),
          pltpu.VMEM((B,), jnp.int32),
          pltpu.VMEM((B, Wd), jnp.float32),
          pltpu.VMEM_SHARED((N_PAD, Wd), jnp.float32),
          pltpu.SemaphoreType.DMA,
      ],
      compiler_params=_sc_params,
  )
  def spmm(w_hbm, src_hbm, dst_hbm, z_hbm, out_hbm,
           src_v, dst_v, rows_v, acc, sem):
    c = lax.axis_index("c")
    s = lax.axis_index("s")
    wid = s * NC + c
    r0 = s * ROWS_PER_SUB
    pltpu.sync_copy(z_hbm.at[pl.ds(r0, ROWS_PER_SUB)],
                    acc.at[pl.ds(r0, ROWS_PER_SUB)])
    plsc.subcore_barrier()

    nj = jnp.where(wid < NCHUNK - (NCHUNK // NT) * NT,
                   NCHUNK // NT + 1, NCHUNK // NT)

    @pl.loop(0, nj)
    def _(j):
      base = (wid + j * NT) * B
      pltpu.sync_copy(src_hbm.at[pl.ds(base, B)], src_v)
      pltpu.sync_copy(dst_hbm.at[pl.ds(base, B)], dst_v)
      pltpu.async_copy(w_hbm.at[src_v], rows_v, sem).wait()
      pltpu.sync_copy(rows_v, acc.at[dst_v], add=True)

    plsc.subcore_barrier()
    pltpu.sync_copy(acc.at[pl.ds(r0, ROWS_PER_SUB)],
                    out_hbm.at[c, pl.ds(r0, ROWS_PER_SUB)])

  return spmm


_spmm_h = _make_spmm(H)
_spmm_c = _make_spmm(CP)


@functools.partial(
    pl.kernel,
    out_type=jax.ShapeDtypeStruct((NC, N_PAD, 16), jnp.float32),
    mesh=_mesh,
    scratch_types=[
        pltpu.VMEM((B,), jnp.int32),
        pltpu.VMEM((B, 16), jnp.float32),
        pltpu.VMEM_SHARED((N_PAD, 16), jnp.float32),
        pltpu.SemaphoreType.DMA,
    ],
    compiler_params=_sc_params,
)
def _sc_degree(dst_hbm, z_hbm, ones_hbm, out_hbm, dst_v, ones_v, acc, sem):
  c = lax.axis_index("c")
  s = lax.axis_index("s")
  wid = s * NC + c
  r0 = s * ROWS_PER_SUB
  pltpu.sync_copy(z_hbm.at[pl.ds(r0, ROWS_PER_SUB)],
                  acc.at[pl.ds(r0, ROWS_PER_SUB)])
  pltpu.sync_copy(ones_hbm, ones_v)
  plsc.subcore_barrier()

  nj = jnp.where(wid < NCHUNK - (NCHUNK // NT) * NT,
                 NCHUNK // NT + 1, NCHUNK // NT)

  @pl.loop(0, nj)
  def _(j):
    base = (wid + j * NT) * B
    pltpu.sync_copy(dst_hbm.at[pl.ds(base, B)], dst_v)
    pltpu.sync_copy(ones_v, acc.at[dst_v], add=True)

  plsc.subcore_barrier()
  pltpu.sync_copy(acc.at[pl.ds(r0, ROWS_PER_SUB)],
                  out_hbm.at[c, pl.ds(r0, ROWS_PER_SUB)])


def _prep_body(parts_ref, f_ref, w1_ref, b1_ref, q_ref, n_ref, c1_ref):
  p = parts_ref[...]
  deg = p[0, :N, 0] + p[1, :N, 0] + 1.0
  q_ref[...] = (1.0 / deg)[:, None]
  nrm = lax.rsqrt(deg)
  n_ref[...] = nrm[:, None]
  x = jnp.dot(f_ref[...], w1_ref[...], preferred_element_type=jnp.float32)
  c1_ref[...] = nrm[:, None] * (x + b1_ref[...])


def _update_body(parts_ref, w_ref, q_ref, c_ref, o_ref):
  p = parts_ref[...]
  w = w_ref[...]
  o_ref[...] = q_ref[...] * (p[0, :N] + p[1, :N] + w) - w + c_ref[...]


def _mlp2_body(w_ref, w2_ref, b2_ref, n_ref, o_ref):
  x = jnp.dot(w_ref[...], w2_ref[...], preferred_element_type=jnp.float32)
  o_ref[...] = x + n_ref[...] * b2_ref[...]


def _final_body(parts_ref, w_ref, q_ref, c_ref, n_ref, o_ref):
  p = parts_ref[...]
  w = w_ref[...]
  o_ref[...] = (q_ref[...] * (p[0, :N] + p[1, :N] + w) - w
                + c_ref[...]) / n_ref[...]


def kernel(features, edge_index, W1, b1, W2, b2):
  ei = edge_index.astype(jnp.int32)
  src = ei[0]
  dst = ei[1]
  zH = jnp.zeros((N_PAD, H), jnp.float32)
  zC = jnp.zeros((N_PAD, CP), jnp.float32)
  z16 = jnp.zeros((N_PAD, 16), jnp.float32)
  ones = jnp.ones((B, 16), jnp.float32)
  W2p = jnp.pad(W2, ((0, 0), (0, CP - C)))
  b2p = jnp.pad(b2, ((0, CP - C),))

  deg_parts = _sc_degree(dst, z16, ones)

  q, nrm, c1 = pl.pallas_call(
      _prep_body,
      out_shape=[
          jax.ShapeDtypeStruct((N, 1), jnp.float32),
          jax.ShapeDtypeStruct((N, 1), jnp.float32),
          jax.ShapeDtypeStruct((N, H), jnp.float32),
      ],
  )(deg_parts, features, W1, b1)

  update_h = pl.pallas_call(
      _update_body, out_shape=jax.ShapeDtypeStruct((N, H), jnp.float32))
  update_c = pl.pallas_call(
      _update_body, out_shape=jax.ShapeDtypeStruct((N, CP), jnp.float32))

  w = c1
  for _ in range(4):
    parts = _spmm_h(w, src, dst, zH)
    w = update_h(parts, w, q, c1)

  c2 = pl.pallas_call(
      _mlp2_body, out_shape=jax.ShapeDtypeStruct((N, CP), jnp.float32))(
          w, W2p, b2p, nrm)

  v = c2
  for _ in range(3):
    parts = _spmm_c(v, src, dst, zC)
    v = update_c(parts, v, q, c2)

  parts = _spmm_c(v, src, dst, zC)
  out = pl.pallas_call(
      _final_body, out_shape=jax.ShapeDtypeStruct((N, CP), jnp.float32))(
          parts, v, q, c2, nrm)

  return out[:, :C]
